# Initial kernel scaffold; baseline (speedup 1.0000x reference)
import jax
import jax.numpy as jnp
from jax.experimental import pallas as pl


def kernel(x, codebook):
    raise NotImplementedError("write your pallas kernel here")


# speedup vs baseline: 1.5664x; 1.5664x over previous
import functools

import jax
import jax.numpy as jnp
from jax import lax
from jax.experimental import pallas as pl
from jax.experimental.pallas import tpu as pltpu
from jax.experimental.pallas import tpu_sc as plsc

_INFO = plsc.get_sparse_core_info()
_NC, _NS = _INFO.num_cores, _INFO.num_subcores
_NW = _NC * _NS

_B = 16 * 1024
_D = 64
_BPW = _B // _NW
_CHUNK = 128
_NCHUNK = _BPW // _CHUNK

_mesh = plsc.VectorSubcoreMesh(core_axis_name="c", subcore_axis_name="s")


@functools.partial(
    pl.kernel,
    mesh=_mesh,
    out_type=jax.ShapeDtypeStruct((_B, _D), jnp.float32),
    scratch_types=[
        pltpu.VMEM((_NCHUNK, _CHUNK), jnp.int32),
        pltpu.VMEM((_BPW, _D), jnp.float32),
        pltpu.SemaphoreType.DMA,
    ],
    compiler_params=pltpu.CompilerParams(use_tc_tiling_on_sc=False),
)
def _gather_kernel(table_hbm, idx_hbm, out_hbm, idx_v, rows_v, sem):
    wid = lax.axis_index("s") * _NC + lax.axis_index("c")
    base = wid * _BPW
    pltpu.sync_copy(idx_hbm.at[wid], idx_v)
    for j in range(_NCHUNK):
        pltpu.async_copy(
            table_hbm.at[idx_v.at[j]],
            rows_v.at[pl.ds(j * _CHUNK, _CHUNK)],
            sem,
        )
    for j in range(_NCHUNK):
        pltpu.make_async_copy(
            table_hbm.at[idx_v.at[j]],
            rows_v.at[pl.ds(j * _CHUNK, _CHUNK)],
            sem,
        ).wait()
    pltpu.sync_copy(rows_v, out_hbm.at[pl.ds(base, _BPW)])


def kernel(x, codebook):
    idx = x.reshape(_NW, _NCHUNK, _CHUNK).astype(jnp.int32)
    flat = _gather_kernel(codebook, idx)
    return flat.reshape(x.shape + (_D,))


# speedup vs baseline: 1.6936x; 1.0812x over previous
import functools

import jax
import jax.numpy as jnp
from jax import lax
from jax.experimental import pallas as pl
from jax.experimental.pallas import tpu as pltpu
from jax.experimental.pallas import tpu_sc as plsc

_INFO = plsc.get_sparse_core_info()
_NC, _NS = _INFO.num_cores, _INFO.num_subcores
_NW = _NC * _NS

_B = 16 * 1024
_D = 64
_BPW = _B // _NW
_CHUNK = 128
_NCHUNK = _BPW // _CHUNK

_mesh = plsc.VectorSubcoreMesh(core_axis_name="c", subcore_axis_name="s")


_V = 512
_RPS = _V // _NS


@functools.partial(
    pl.kernel,
    mesh=_mesh,
    out_type=jax.ShapeDtypeStruct((_B, _D), jnp.float32),
    scratch_types=[
        pltpu.VMEM((_NCHUNK, _CHUNK), jnp.int32),
        pltpu.VMEM((_BPW, _D), jnp.float32),
        pltpu.VMEM_SHARED((_V, _D), jnp.float32),
        pltpu.SemaphoreType.DMA((_NCHUNK,)),
        pltpu.SemaphoreType.DMA((_NCHUNK,)),
    ],
    compiler_params=pltpu.CompilerParams(use_tc_tiling_on_sc=False),
)
def _gather_kernel(table_hbm, idx_hbm, out_hbm, idx_v, rows_v, table_s, gsem, wsem):
    cid = lax.axis_index("c")
    sid = lax.axis_index("s")
    wid = sid * _NC + cid
    base = wid * _BPW
    pltpu.sync_copy(
        table_hbm.at[pl.ds(sid * _RPS, _RPS)],
        table_s.at[pl.ds(sid * _RPS, _RPS)],
    )
    pltpu.sync_copy(idx_hbm.at[wid], idx_v)
    plsc.subcore_barrier()
    for j in range(_NCHUNK):
        pltpu.async_copy(
            table_s.at[idx_v.at[j]],
            rows_v.at[pl.ds(j * _CHUNK, _CHUNK)],
            gsem.at[j],
        )
    for j in range(_NCHUNK):
        pltpu.make_async_copy(
            table_s.at[idx_v.at[j]],
            rows_v.at[pl.ds(j * _CHUNK, _CHUNK)],
            gsem.at[j],
        ).wait()
        pltpu.async_copy(
            rows_v.at[pl.ds(j * _CHUNK, _CHUNK)],
            out_hbm.at[pl.ds(base + j * _CHUNK, _CHUNK)],
            wsem.at[j],
        )
    for j in range(_NCHUNK):
        pltpu.make_async_copy(
            rows_v.at[pl.ds(j * _CHUNK, _CHUNK)],
            out_hbm.at[pl.ds(base + j * _CHUNK, _CHUNK)],
            wsem.at[j],
        ).wait()


def kernel(x, codebook):
    idx = x.reshape(_NW, _NCHUNK, _CHUNK).astype(jnp.int32)
    flat = _gather_kernel(codebook, idx)
    return flat.reshape(x.shape + (_D,))


# speedup vs baseline: 1.6967x; 1.0018x over previous
import functools

import jax
import jax.numpy as jnp
from jax import lax
from jax.experimental import pallas as pl
from jax.experimental.pallas import tpu as pltpu
from jax.experimental.pallas import tpu_sc as plsc

_INFO = plsc.get_sparse_core_info()
_NC, _NS = _INFO.num_cores, _INFO.num_subcores
_NW = _NC * _NS

_B = 16 * 1024
_D = 64
_BPW = _B // _NW
_CHUNK = 128
_NCHUNK = _BPW // _CHUNK

_mesh = plsc.VectorSubcoreMesh(core_axis_name="c", subcore_axis_name="s")


_V = 512
_RPS = _V // _NS


@functools.partial(
    pl.kernel,
    mesh=_mesh,
    out_type=jax.ShapeDtypeStruct((_B, _D), jnp.float32),
    scratch_types=[
        pltpu.VMEM((_NCHUNK, _CHUNK), jnp.int32),
        pltpu.VMEM((_BPW, _D), jnp.float32),
        pltpu.VMEM_SHARED((_V, _D), jnp.float32),
        pltpu.SemaphoreType.DMA((_NCHUNK,)),
        pltpu.SemaphoreType.DMA((_NCHUNK,)),
    ],
    compiler_params=pltpu.CompilerParams(
        use_tc_tiling_on_sc=False,
        skip_device_barrier=True,
        disable_bounds_checks=True,
        disable_semaphore_checks=True,
    ),
)
def _gather_kernel(table_hbm, idx_hbm, out_hbm, idx_v, rows_v, table_s, gsem, wsem):
    cid = lax.axis_index("c")
    sid = lax.axis_index("s")
    wid = sid * _NC + cid
    base = wid * _BPW
    pltpu.sync_copy(
        table_hbm.at[pl.ds(sid * _RPS, _RPS)],
        table_s.at[pl.ds(sid * _RPS, _RPS)],
    )
    pltpu.sync_copy(idx_hbm.at[wid], idx_v)
    plsc.subcore_barrier()
    for j in range(_NCHUNK):
        pltpu.async_copy(
            table_s.at[idx_v.at[j]],
            rows_v.at[pl.ds(j * _CHUNK, _CHUNK)],
            gsem.at[j],
        )
    for j in range(_NCHUNK):
        pltpu.make_async_copy(
            table_s.at[idx_v.at[j]],
            rows_v.at[pl.ds(j * _CHUNK, _CHUNK)],
            gsem.at[j],
        ).wait()
        pltpu.async_copy(
            rows_v.at[pl.ds(j * _CHUNK, _CHUNK)],
            out_hbm.at[pl.ds(base + j * _CHUNK, _CHUNK)],
            wsem.at[j],
        )
    for j in range(_NCHUNK):
        pltpu.make_async_copy(
            rows_v.at[pl.ds(j * _CHUNK, _CHUNK)],
            out_hbm.at[pl.ds(base + j * _CHUNK, _CHUNK)],
            wsem.at[j],
        ).wait()


def kernel(x, codebook):
    idx = x.reshape(_NW, _NCHUNK, _CHUNK).astype(jnp.int32)
    flat = _gather_kernel(codebook, idx)
    return flat.reshape(x.shape + (_D,))


# speedup vs baseline: 1.6998x; 1.0018x over previous
import functools

import jax
import jax.numpy as jnp
from jax import lax
from jax.experimental import pallas as pl
from jax.experimental.pallas import tpu as pltpu
from jax.experimental.pallas import tpu_sc as plsc

_INFO = plsc.get_sparse_core_info()
_NC, _NS = _INFO.num_cores, _INFO.num_subcores
_NW = _NC * _NS

_BATCH = 16
_SEQ = 1024
_B = _BATCH * _SEQ
_D = 64
_BPW = _B // _NW
_CHUNK = 128
_NCHUNK = _BPW // _CHUNK
_WPB = _SEQ // _BPW

_V = 512
_RPS = _V // _NS

_mesh = plsc.VectorSubcoreMesh(core_axis_name="c", subcore_axis_name="s")


@functools.partial(
    pl.kernel,
    mesh=_mesh,
    out_type=jax.ShapeDtypeStruct((_BATCH, _SEQ, _D), jnp.float32),
    scratch_types=[
        pltpu.VMEM((_BPW,), jnp.int32),
        pltpu.VMEM((_BPW, _D), jnp.float32),
        pltpu.VMEM_SHARED((_V, _D), jnp.float32),
        pltpu.SemaphoreType.DMA((_NCHUNK,)),
        pltpu.SemaphoreType.DMA((_NCHUNK,)),
    ],
    compiler_params=pltpu.CompilerParams(use_tc_tiling_on_sc=False),
)
def _gather_kernel(table_hbm, idx_hbm, out_hbm, idx_v, rows_v, table_s, gsem, wsem):
    cid = lax.axis_index("c")
    sid = lax.axis_index("s")
    wid = sid * _NC + cid
    row = wid // _WPB
    off = (wid % _WPB) * _BPW
    pltpu.sync_copy(
        table_hbm.at[pl.ds(sid * _RPS, _RPS)],
        table_s.at[pl.ds(sid * _RPS, _RPS)],
    )
    pltpu.sync_copy(idx_hbm.at[row, pl.ds(off, _BPW)], idx_v)
    plsc.subcore_barrier()
    for j in range(_NCHUNK):
        pltpu.async_copy(
            table_s.at[idx_v.at[pl.ds(j * _CHUNK, _CHUNK)]],
            rows_v.at[pl.ds(j * _CHUNK, _CHUNK)],
            gsem.at[j],
        )
    for j in range(_NCHUNK):
        pltpu.make_async_copy(
            table_s.at[idx_v.at[pl.ds(j * _CHUNK, _CHUNK)]],
            rows_v.at[pl.ds(j * _CHUNK, _CHUNK)],
            gsem.at[j],
        ).wait()
        pltpu.async_copy(
            rows_v.at[pl.ds(j * _CHUNK, _CHUNK)],
            out_hbm.at[row, pl.ds(off + j * _CHUNK, _CHUNK)],
            wsem.at[j],
        )
    for j in range(_NCHUNK):
        pltpu.make_async_copy(
            rows_v.at[pl.ds(j * _CHUNK, _CHUNK)],
            out_hbm.at[row, pl.ds(off + j * _CHUNK, _CHUNK)],
            wsem.at[j],
        ).wait()


def kernel(x, codebook):
    return _gather_kernel(codebook, x.astype(jnp.int32))


# speedup vs baseline: 1.9691x; 1.1584x over previous
import functools

import jax
import jax.numpy as jnp
from jax import lax
from jax.experimental import pallas as pl
from jax.experimental.pallas import tpu as pltpu
from jax.experimental.pallas import tpu_sc as plsc

_INFO = plsc.get_sparse_core_info()
_NC, _NS = _INFO.num_cores, _INFO.num_subcores
_NW = _NC * _NS

_BATCH = 16
_SEQ = 1024
_B = _BATCH * _SEQ
_D = 64
_DW = 2 * _D
_L = 16
_BPW = _B // _NW
_CHUNK = 128
_NCHUNK = _BPW // _CHUNK
_WPB = _SEQ // _BPW

_V = 512
_RPS = _V // _NS

_mesh = plsc.VectorSubcoreMesh(core_axis_name="c", subcore_axis_name="s")


@functools.partial(
    pl.kernel,
    mesh=_mesh,
    out_type=jax.ShapeDtypeStruct((_BATCH, _SEQ, _D), jnp.float32),
    scratch_types=[
        pltpu.VMEM((_BPW,), jnp.int32),
        pltpu.VMEM((2, _CHUNK, _DW), jnp.float32),
        pltpu.VMEM((_BPW, _D), jnp.float32),
        pltpu.VMEM_SHARED((_V, _DW), jnp.float32),
        pltpu.SemaphoreType.DMA((_NCHUNK,)),
        pltpu.SemaphoreType.DMA((_NCHUNK,)),
    ],
)
def _gather_kernel(table_hbm, idx_hbm, out_hbm, idx_v, rows_v, outc_v, table_s, gsem, wsem):
    cid = lax.axis_index("c")
    sid = lax.axis_index("s")
    wid = sid * _NC + cid
    row = wid // _WPB
    off = (wid % _WPB) * _BPW
    pltpu.sync_copy(
        table_hbm.at[pl.ds(sid * _RPS, _RPS)],
        table_s.at[pl.ds(sid * _RPS, _RPS)],
    )
    pltpu.sync_copy(idx_hbm.at[row, pl.ds(off, _BPW)], idx_v)
    plsc.subcore_barrier()
    for j in range(2):
        pltpu.async_copy(
            table_s.at[idx_v.at[pl.ds(j * _CHUNK, _CHUNK)]],
            rows_v.at[j],
            gsem.at[j],
        )
    for j in range(_NCHUNK):
        slot = j % 2
        pltpu.make_async_copy(
            table_s.at[idx_v.at[pl.ds(j * _CHUNK, _CHUNK)]],
            rows_v.at[slot],
            gsem.at[j],
        ).wait()

        def _compact(i, carry, j=j, slot=slot):
            for c in range(_D // _L):
                outc_v[j * _CHUNK + i, pl.ds(c * _L, _L)] = rows_v[slot, i, pl.ds(c * _L, _L)]
            return carry

        lax.fori_loop(0, _CHUNK, _compact, 0)
        if j + 2 < _NCHUNK:
            pltpu.async_copy(
                table_s.at[idx_v.at[pl.ds((j + 2) * _CHUNK, _CHUNK)]],
                rows_v.at[slot],
                gsem.at[j + 2],
            )
        pltpu.async_copy(
            outc_v.at[pl.ds(j * _CHUNK, _CHUNK)],
            out_hbm.at[row, pl.ds(off + j * _CHUNK, _CHUNK)],
            wsem.at[j],
        )
    for j in range(_NCHUNK):
        pltpu.make_async_copy(
            outc_v.at[pl.ds(j * _CHUNK, _CHUNK)],
            out_hbm.at[row, pl.ds(off + j * _CHUNK, _CHUNK)],
            wsem.at[j],
        ).wait()


def kernel(x, codebook):
    wide = jnp.concatenate([codebook, codebook], axis=1)
    return _gather_kernel(wide, x.astype(jnp.int32))
